# baseline (speedup 1.0000x reference)
import functools

import jax
import jax.numpy as jnp
from jax import lax
from jax.experimental import pallas as pl
from jax.experimental.pallas import tpu as pltpu
from jax.experimental.pallas import tpu_sc as plsc

WINDOW = 252

_NC = 2
_NS = 16
_NW = _NC * _NS
_LANE = 128
_EMIT_BLK = 8


def _emit_chunk(t_rows):
    b = _EMIT_BLK
    return (((t_rows + _NW - 1) // _NW + b - 1) // b) * b


def _sc_gather(data, tail_pad, idx):
    t_rows, n_cols = data.shape
    n_main = (n_cols // _LANE) * _LANE
    tv = idx.shape[0]
    blk = 16
    chunk = (((tv + _NW - 1) // _NW + blk - 1) // blk) * blk
    mesh = plsc.VectorSubcoreMesh(core_axis_name="c", subcore_axis_name="s")

    @functools.partial(
        pl.kernel,
        out_type=jax.ShapeDtypeStruct((tv, n_main + _LANE), jnp.float32),
        mesh=mesh,
        scratch_types=[
            pltpu.VMEM((blk,), jnp.int32),
            pltpu.VMEM((blk, n_main), jnp.float32),
            pltpu.VMEM((blk, _LANE), jnp.float32),
            pltpu.SemaphoreType.DMA,
            pltpu.SemaphoreType.DMA,
        ],
    )
    def k(data_hbm, tail_hbm, idx_hbm, out_hbm, idx_v, rows_m, rows_t, s1, s2):
        wid = lax.axis_index("s") * _NC + lax.axis_index("c")
        start = wid * chunk
        count = jnp.clip(tv - start, 0, chunk)

        def body(j, carry):
            base = start + j * blk
            pltpu.sync_copy(idx_hbm.at[pl.ds(base, blk)], idx_v)
            cm = pltpu.async_copy(data_hbm.at[idx_v, pl.ds(0, n_main)], rows_m, s1)
            ct = pltpu.async_copy(tail_hbm.at[idx_v], rows_t, s2)
            cm.wait()
            ct.wait()
            pltpu.sync_copy(rows_m, out_hbm.at[pl.ds(base, blk), pl.ds(0, n_main)])
            pltpu.sync_copy(rows_t, out_hbm.at[pl.ds(base, blk), pl.ds(n_main, _LANE)])
            return carry

        lax.fori_loop(0, count // blk, body, 0)

    return k(data, tail_pad, idx)


def _tc_rolling_mean(valid, idx, t_ext, pa_len):
    tv, n_pad = valid.shape
    w = WINDOW
    tr = 240
    nch = tv // tr
    s = w - tr
    cb = 512
    ncb = n_pad // cb

    def body(v_ref, idx_ref, o_ref, pa_ref):
        @pl.when(pl.program_id(0) == 0)
        def _():
            def init(r, carry):
                pa_ref[r] = tv + (r & 7)
                return carry

            lax.fori_loop(0, pa_len, init, 0)

            def scat(t, carry):
                pa_ref[idx_ref[t]] = t
                return carry

            lax.fori_loop(0, tv, scat, 0)

        nan = jnp.float32(jnp.nan)
        inv_w = jnp.float32(1.0 / w)
        tri = (
            lax.broadcasted_iota(jnp.int32, (tr, tr), 0)
            >= lax.broadcasted_iota(jnp.int32, (tr, tr), 1)
        ).astype(jnp.float32)
        carry = jnp.zeros((1, cb), jnp.float32)
        prev1 = None
        prev2 = None
        for k in range(nch):
            a = k * tr
            chunk = v_ref[a : a + tr, :]
            pre = lax.dot_general(
                tri,
                chunk,
                (((1,), (0,)), ((), ())),
                precision=lax.Precision.HIGHEST,
                preferred_element_type=jnp.float32,
            )
            cs = pre + carry
            carry = cs[tr - 1 : tr, :]
            if k == 0:
                sh = jnp.zeros((tr, cb), jnp.float32)
            elif k == 1:
                sh = jnp.concatenate(
                    [jnp.zeros((s, cb), jnp.float32), prev1[: tr - s, :]], axis=0
                )
            else:
                sh = jnp.concatenate(
                    [prev2[tr - s :, :], prev1[: tr - s, :]], axis=0
                )
            val = (cs - sh) * inv_w
            gid = a + lax.broadcasted_iota(jnp.int32, (tr, cb), 0)
            o_ref[a : a + tr, :] = jnp.where(gid >= w - 1, val, nan)
            prev2 = prev1
            prev1 = cs
        o_ref[tv:t_ext, :] = jnp.full((t_ext - tv, cb), nan)

    return pl.pallas_call(
        body,
        grid=(ncb,),
        in_specs=[
            pl.BlockSpec((tv, cb), lambda j: (0, j)),
            pl.BlockSpec(memory_space=pltpu.SMEM),
        ],
        out_specs=[
            pl.BlockSpec((t_ext, cb), lambda j: (0, j)),
            pl.BlockSpec(memory_space=pltpu.SMEM),
        ],
        out_shape=[
            jax.ShapeDtypeStruct((t_ext, n_pad), jnp.float32),
            jax.ShapeDtypeStruct((pa_len,), jnp.int32),
        ],
    )(valid, idx)


def _sc_emit_main(rolled, pa_ext, t_rows, n_cols):
    n_main = (n_cols // _LANE) * _LANE
    blk = _EMIT_BLK
    chunk = _emit_chunk(t_rows)
    mesh = plsc.VectorSubcoreMesh(core_axis_name="c", subcore_axis_name="s")

    @functools.partial(
        pl.kernel,
        out_type=jax.ShapeDtypeStruct((t_rows, n_cols), jnp.float32),
        mesh=mesh,
        scratch_types=[
            pltpu.VMEM((chunk,), jnp.int32),
            pltpu.VMEM((blk, n_main), jnp.float32),
            pltpu.VMEM((blk, n_main), jnp.float32),
            pltpu.SemaphoreType.DMA,
            pltpu.SemaphoreType.DMA,
        ],
    )
    def k(rolled_hbm, pa_hbm, out_hbm, pa_all, rows_a, rows_b, sem_a, sem_b):
        wid = lax.axis_index("s") * _NC + lax.axis_index("c")
        start = wid * chunk
        count = jnp.clip(t_rows - start, 0, chunk)
        nblk = count // blk
        bufs = ((rows_a, sem_a), (rows_b, sem_b))

        def gather(j, par):
            rv, sem = bufs[par]
            pltpu.async_copy(
                rolled_hbm.at[pa_all.at[pl.ds(j * blk, blk)], pl.ds(0, n_main)],
                rv,
                sem,
            )

        def drain_write(j, par):
            rv, sem = bufs[par]
            pltpu.make_async_copy(
                rolled_hbm.at[pa_all.at[pl.ds(0, blk)], pl.ds(0, n_main)], rv, sem
            ).wait()
            pltpu.sync_copy(
                rv, out_hbm.at[pl.ds(start + j * blk, blk), pl.ds(0, n_main)]
            )

        @pl.when(nblk > 0)
        def _():
            pltpu.sync_copy(pa_hbm.at[pl.ds(start, chunk)], pa_all)
            gather(0, 0)

            def pair(j2, carry):
                j = j2 * 2

                @pl.when(j + 1 < nblk)
                def _():
                    gather(j + 1, 1)

                drain_write(j, 0)

                @pl.when(j + 2 < nblk)
                def _():
                    gather(j + 2, 0)

                @pl.when(j + 1 < nblk)
                def _():
                    drain_write(j + 1, 1)

                return carry

            lax.fori_loop(0, (nblk + 1) // 2, pair, 0)

    return k(rolled, pa_ext)


def _tc_emit_tail(rolled, pa_ext, out_part):
    t_rows, n_cols = out_part.shape
    n_main = (n_cols // _LANE) * _LANE
    t_ext = rolled.shape[0]
    jblk = n_main // _LANE

    def body(pa_ref, slab_ref, _unused, o_ref):
        def grp(i, carry):
            rows = [
                slab_ref[pl.ds(pa_ref[i * 8 + k], 1), :] for k in range(8)
            ]
            o_ref[pl.ds(i * 8, 8), :] = jnp.concatenate(rows, axis=0)
            return carry

        lax.fori_loop(0, t_rows // 8, grp, 0)

    return pl.pallas_call(
        body,
        grid=(1,),
        in_specs=[
            pl.BlockSpec(memory_space=pltpu.SMEM),
            pl.BlockSpec((t_ext, _LANE), lambda j: (0, jblk)),
            pl.BlockSpec(memory_space=pl.ANY),
        ],
        out_specs=pl.BlockSpec((t_rows, _LANE), lambda j: (0, jblk)),
        out_shape=jax.ShapeDtypeStruct((t_rows, n_cols), jnp.float32),
        input_output_aliases={2: 0},
    )(pa_ext, rolled, out_part)


def kernel(data, indices, mask):
    del mask
    t_rows, n_cols = data.shape
    n_main = (n_cols // _LANE) * _LANE
    tv = indices.shape[0]

    idx = jnp.where(indices == -1, 0, indices).astype(jnp.int32)
    t_ext = tv + 8
    pa_len = _NW * _emit_chunk(t_rows)
    tail_pad = jnp.pad(data[:, n_main:], ((0, 0), (0, _LANE - (n_cols - n_main))))

    valid = _sc_gather(data, tail_pad, idx)
    rolled, pa_ext = _tc_rolling_mean(valid, idx, t_ext, pa_len)
    out_part = _sc_emit_main(rolled, pa_ext, t_rows, n_cols)
    return _tc_emit_tail(rolled, pa_ext, out_part)


# speedup vs baseline: 1.1147x; 1.1147x over previous
import functools

import jax
import jax.numpy as jnp
from jax import lax
from jax.experimental import pallas as pl
from jax.experimental.pallas import tpu as pltpu
from jax.experimental.pallas import tpu_sc as plsc

WINDOW = 252

_NC = 2
_NS = 16
_NW = _NC * _NS
_LANE = 128


def _sc_gather(data, tail_pad, idx):
    t_rows, n_cols = data.shape
    n_main = (n_cols // _LANE) * _LANE
    tv = idx.shape[0]
    blk = 16
    chunk = (((tv + _NW - 1) // _NW + blk - 1) // blk) * blk
    mesh = plsc.VectorSubcoreMesh(core_axis_name="c", subcore_axis_name="s")

    @functools.partial(
        pl.kernel,
        out_type=jax.ShapeDtypeStruct((tv, n_main + _LANE), jnp.float32),
        mesh=mesh,
        scratch_types=[
            pltpu.VMEM((blk,), jnp.int32),
            pltpu.VMEM((blk, n_main), jnp.float32),
            pltpu.VMEM((blk, _LANE), jnp.float32),
            pltpu.SemaphoreType.DMA,
            pltpu.SemaphoreType.DMA,
        ],
    )
    def k(data_hbm, tail_hbm, idx_hbm, out_hbm, idx_v, rows_m, rows_t, s1, s2):
        wid = lax.axis_index("s") * _NC + lax.axis_index("c")
        start = wid * chunk
        count = jnp.clip(tv - start, 0, chunk)

        def body(j, carry):
            base = start + j * blk
            pltpu.sync_copy(idx_hbm.at[pl.ds(base, blk)], idx_v)
            cm = pltpu.async_copy(data_hbm.at[idx_v, pl.ds(0, n_main)], rows_m, s1)
            ct = pltpu.async_copy(tail_hbm.at[idx_v], rows_t, s2)
            cm.wait()
            ct.wait()
            pltpu.sync_copy(rows_m, out_hbm.at[pl.ds(base, blk), pl.ds(0, n_main)])
            pltpu.sync_copy(rows_t, out_hbm.at[pl.ds(base, blk), pl.ds(n_main, _LANE)])
            return carry

        lax.fori_loop(0, count // blk, body, 0)

    return k(data, tail_pad, idx)


def _tc_rolling_mean(valid, t_ext):
    tv, n_pad = valid.shape
    w = WINDOW
    tr = 240
    nch = tv // tr
    s = w - tr
    cb = 512
    ncb = n_pad // cb

    def body(v_ref, o_ref):
        nan = jnp.float32(jnp.nan)
        inv_w = jnp.float32(1.0 / w)
        tri = (
            lax.broadcasted_iota(jnp.int32, (tr, tr), 0)
            >= lax.broadcasted_iota(jnp.int32, (tr, tr), 1)
        ).astype(jnp.float32)
        carry = jnp.zeros((1, cb), jnp.float32)
        prev1 = None
        prev2 = None
        for k in range(nch):
            a = k * tr
            chunk = v_ref[a : a + tr, :]
            pre = lax.dot_general(
                tri,
                chunk,
                (((1,), (0,)), ((), ())),
                precision=lax.Precision.DEFAULT,
                preferred_element_type=jnp.float32,
            )
            cs = pre + carry
            carry = cs[tr - 1 : tr, :]
            if k == 0:
                sh = jnp.zeros((tr, cb), jnp.float32)
            elif k == 1:
                sh = jnp.concatenate(
                    [jnp.zeros((s, cb), jnp.float32), prev1[: tr - s, :]], axis=0
                )
            else:
                sh = jnp.concatenate(
                    [prev2[tr - s :, :], prev1[: tr - s, :]], axis=0
                )
            val = (cs - sh) * inv_w
            gid = a + lax.broadcasted_iota(jnp.int32, (tr, cb), 0)
            o_ref[a : a + tr, :] = jnp.where(gid >= w - 1, val, nan)
            prev2 = prev1
            prev1 = cs
        o_ref[tv:t_ext, :] = jnp.full((t_ext - tv, cb), nan)

    return pl.pallas_call(
        body,
        grid=(ncb,),
        in_specs=[pl.BlockSpec((tv, cb), lambda j: (0, j))],
        out_specs=pl.BlockSpec((t_ext, cb), lambda j: (0, j)),
        out_shape=jax.ShapeDtypeStruct((t_ext, n_pad), jnp.float32),
    )(valid)


def _sc_emit_main(rolled, pa_ext, t_rows, n_cols):
    n_main = (n_cols // _LANE) * _LANE
    blk = 16
    chunk = (((t_rows + _NW - 1) // _NW + blk - 1) // blk) * blk
    pa_ext = jnp.pad(pa_ext, (0, _NW * chunk - t_rows))
    mesh = plsc.VectorSubcoreMesh(core_axis_name="c", subcore_axis_name="s")

    @functools.partial(
        pl.kernel,
        out_type=jax.ShapeDtypeStruct((t_rows, n_cols), jnp.float32),
        mesh=mesh,
        scratch_types=[
            pltpu.VMEM((chunk,), jnp.int32),
            pltpu.VMEM((blk, n_main), jnp.float32),
            pltpu.SemaphoreType.DMA,
        ],
    )
    def k(rolled_hbm, pa_hbm, out_hbm, pa_all, rows_v, sem):
        wid = lax.axis_index("s") * _NC + lax.axis_index("c")
        start = wid * chunk
        count = jnp.clip(t_rows - start, 0, chunk)
        nblk = count // blk

        @pl.when(nblk > 0)
        def _():
            pltpu.sync_copy(pa_hbm.at[pl.ds(start, chunk)], pa_all)

            def body(j, carry):
                pltpu.async_copy(
                    rolled_hbm.at[pa_all.at[pl.ds(j * blk, blk)], pl.ds(0, n_main)],
                    rows_v,
                    sem,
                ).wait()
                pltpu.sync_copy(
                    rows_v,
                    out_hbm.at[pl.ds(start + j * blk, blk), pl.ds(0, n_main)],
                )
                return carry

            lax.fori_loop(0, nblk, body, 0)

    return k(rolled, pa_ext)


def _tc_emit_tail(rolled, pa_ext, out_part):
    t_rows, n_cols = out_part.shape
    n_main = (n_cols // _LANE) * _LANE
    t_ext = rolled.shape[0]
    jblk = n_main // _LANE

    def body(pa_ref, slab_ref, _unused, o_ref):
        def grp(i, carry):
            rows = [
                slab_ref[pl.ds(pa_ref[i * 8 + k], 1), :] for k in range(8)
            ]
            o_ref[pl.ds(i * 8, 8), :] = jnp.concatenate(rows, axis=0)
            return carry

        lax.fori_loop(0, t_rows // 8, grp, 0)

    return pl.pallas_call(
        body,
        grid=(1,),
        in_specs=[
            pl.BlockSpec(memory_space=pltpu.SMEM),
            pl.BlockSpec((t_ext, _LANE), lambda j: (0, jblk)),
            pl.BlockSpec(memory_space=pl.ANY),
        ],
        out_specs=pl.BlockSpec((t_rows, _LANE), lambda j: (0, jblk)),
        out_shape=jax.ShapeDtypeStruct((t_rows, n_cols), jnp.float32),
        input_output_aliases={2: 0},
    )(pa_ext, rolled, out_part)


def kernel(data, indices, mask):
    del mask
    t_rows, n_cols = data.shape
    n_main = (n_cols // _LANE) * _LANE
    tv = indices.shape[0]

    idx = jnp.where(indices == -1, 0, indices).astype(jnp.int32)
    ar = jnp.arange(t_rows, dtype=jnp.int32)
    last_pos = (
        jnp.full((t_rows,), -1, jnp.int32)
        .at[idx]
        .max(jnp.arange(tv, dtype=jnp.int32))
    )
    t_ext = tv + 8
    nan_row = tv + (ar & 7)
    pa_ext = jnp.where(last_pos >= 0, last_pos, nan_row).astype(jnp.int32)
    tail_pad = jnp.pad(data[:, n_main:], ((0, 0), (0, _LANE - (n_cols - n_main))))

    valid = _sc_gather(data, tail_pad, idx)
    rolled = _tc_rolling_mean(valid, t_ext)
    out_part = _sc_emit_main(rolled, pa_ext, t_rows, n_cols)
    return _tc_emit_tail(rolled, pa_ext, out_part)
